# baseline (speedup 1.0000x reference)
import jax
import jax.numpy as jnp
from jax.experimental import pallas as pl
from jax.experimental.pallas import tpu as pltpu

H = 128
MB = 128
JB = 128
KC = 128


def _pair_kernel(a_ref, b_ref, w1_ref, b1_ref, w2_ref, b2_ref, out_ref):
    a = a_ref[...]
    b = b_ref[...]
    w1 = w1_ref[...]
    ha = jax.lax.dot_general(a, w1[:, :H], (((1,), (1,)), ((), ())),
                             preferred_element_type=jnp.float32)
    ha = ha + b1_ref[...]
    hb = jax.lax.dot_general(b, w1[:, H:], (((1,), (1,)), ((), ())),
                             preferred_element_type=jnp.float32)

    acc = jnp.full((MB, JB), b2_ref[0], dtype=jnp.float32)
    for k0 in range(0, 2 * H, KC):
        t = jnp.maximum(ha[:, None, k0:k0 + KC] + hb[None, :, k0:k0 + KC],
                        0.0)
        w2c = w2_ref[0, k0:k0 + KC]
        acc = acc + jnp.sum(t * w2c[None, None, :], axis=-1)
    out_ref[...] = acc


def kernel(pred_a, pred_b, W1, b1, W2, b2):
    n_m = pred_a.shape[0]
    n_j = pred_b.shape[0]
    b1r = b1.reshape(1, 2 * H)
    grid = (n_m // MB, n_j // JB)
    return pl.pallas_call(
        _pair_kernel,
        grid=grid,
        in_specs=[
            pl.BlockSpec((MB, H), lambda i, j: (i, 0)),
            pl.BlockSpec((JB, H), lambda i, j: (j, 0)),
            pl.BlockSpec((2 * H, 2 * H), lambda i, j: (0, 0)),
            pl.BlockSpec((1, 2 * H), lambda i, j: (0, 0)),
            pl.BlockSpec((1, 2 * H), lambda i, j: (0, 0)),
            pl.BlockSpec(memory_space=pltpu.SMEM),
        ],
        out_specs=pl.BlockSpec((MB, JB), lambda i, j: (i, j)),
        out_shape=jax.ShapeDtypeStruct((n_m, n_j), jnp.float32),
        compiler_params=pltpu.CompilerParams(
            dimension_semantics=("parallel", "arbitrary"),
            vmem_limit_bytes=100 * 1024 * 1024,
        ),
    )(pred_a, pred_b, W1, b1r, W2, b2)


# speedup vs baseline: 1.7830x; 1.7830x over previous
import jax
import jax.numpy as jnp
from jax.experimental import pallas as pl
from jax.experimental.pallas import tpu as pltpu

H = 128
MB = 128
JB = 128
MG = 8


def _pair_kernel(a_ref, b_ref, w1_ref, b1_ref, w2_ref, b2_ref, out_ref):
    a = a_ref[...]
    b = b_ref[...]
    w1 = w1_ref[...]
    ha = jax.lax.dot_general(a, w1[:, :H], (((1,), (1,)), ((), ())),
                             preferred_element_type=jnp.float32)
    ha = ha + b1_ref[...]
    hb = jax.lax.dot_general(b, w1[:, H:], (((1,), (1,)), ((), ())),
                             preferred_element_type=jnp.float32)

    w2 = w2_ref[...]
    b2 = b2_ref[0]
    hab = ha.astype(jnp.bfloat16)
    hbb = hb.astype(jnp.bfloat16)
    w2b = w2.astype(jnp.bfloat16)
    for mg in range(0, MB, MG):
        t = jnp.maximum(hab[mg:mg + MG, None, :] + hbb[None, :, :],
                        jnp.bfloat16(0.0))
        s = jnp.sum(t * w2b[None, :, :], axis=-1,
                    dtype=jnp.bfloat16)
        out_ref[mg:mg + MG, :] = s.astype(jnp.float32) + b2


def kernel(pred_a, pred_b, W1, b1, W2, b2):
    n_m = pred_a.shape[0]
    n_j = pred_b.shape[0]
    b1r = b1.reshape(1, 2 * H)
    grid = (n_m // MB, n_j // JB)
    out = pl.pallas_call(
        _pair_kernel,
        grid=grid,
        in_specs=[
            pl.BlockSpec((MB, H), lambda i, j: (i, 0)),
            pl.BlockSpec((JB, H), lambda i, j: (j, 0)),
            pl.BlockSpec((2 * H, 2 * H), lambda i, j: (0, 0)),
            pl.BlockSpec((1, 2 * H), lambda i, j: (0, 0)),
            pl.BlockSpec((1, 2 * H), lambda i, j: (0, 0)),
            pl.BlockSpec(memory_space=pltpu.SMEM),
        ],
        out_specs=pl.BlockSpec((MB, JB), lambda i, j: (i, j)),
        out_shape=jax.ShapeDtypeStruct((n_m, n_j), jnp.float32),
        compiler_params=pltpu.CompilerParams(
            dimension_semantics=("parallel", "arbitrary"),
            vmem_limit_bytes=100 * 1024 * 1024,
        ),
    )(pred_a, pred_b, W1, b1r, W2, b2)
    return out


# speedup vs baseline: 1.9311x; 1.0831x over previous
import jax
import jax.numpy as jnp
from jax.experimental import pallas as pl
from jax.experimental.pallas import tpu as pltpu

H = 128
MB = 128
JB = 256
MG = 8


def _pair_kernel(a_ref, b_ref, w1_ref, b1_ref, w2_ref, b2_ref, out_ref):
    a = a_ref[...]
    b = b_ref[...]
    w1 = w1_ref[...]
    ha = jax.lax.dot_general(a, w1[:, :H], (((1,), (1,)), ((), ())),
                             preferred_element_type=jnp.float32)
    ha = ha + b1_ref[...]
    hb = jax.lax.dot_general(b, w1[:, H:], (((1,), (1,)), ((), ())),
                             preferred_element_type=jnp.float32)

    w2 = w2_ref[...]
    b2 = b2_ref[0]
    hab = ha.astype(jnp.bfloat16)
    hbb = hb.astype(jnp.bfloat16)
    w2b = w2.astype(jnp.bfloat16)
    for mg in range(0, MB, MG):
        t = jnp.maximum(hab[mg:mg + MG, None, :] + hbb[None, :, :],
                        jnp.bfloat16(0.0))
        s = jnp.sum(t * w2b[None, :, :], axis=-1,
                    dtype=jnp.bfloat16)
        out_ref[mg:mg + MG, :] = s.astype(jnp.float32) + b2


def kernel(pred_a, pred_b, W1, b1, W2, b2):
    n_m = pred_a.shape[0]
    n_j = pred_b.shape[0]
    b1r = b1.reshape(1, 2 * H)
    grid = (n_m // MB, n_j // JB)
    out = pl.pallas_call(
        _pair_kernel,
        grid=grid,
        in_specs=[
            pl.BlockSpec((MB, H), lambda i, j: (i, 0)),
            pl.BlockSpec((JB, H), lambda i, j: (j, 0)),
            pl.BlockSpec((2 * H, 2 * H), lambda i, j: (0, 0)),
            pl.BlockSpec((1, 2 * H), lambda i, j: (0, 0)),
            pl.BlockSpec((1, 2 * H), lambda i, j: (0, 0)),
            pl.BlockSpec(memory_space=pltpu.SMEM),
        ],
        out_specs=pl.BlockSpec((MB, JB), lambda i, j: (i, j)),
        out_shape=jax.ShapeDtypeStruct((n_m, n_j), jnp.float32),
        compiler_params=pltpu.CompilerParams(
            dimension_semantics=("parallel", "arbitrary"),
            vmem_limit_bytes=100 * 1024 * 1024,
        ),
    )(pred_a, pred_b, W1, b1r, W2, b2)
    return out


# speedup vs baseline: 1.9393x; 1.0042x over previous
import jax
import jax.numpy as jnp
from jax.experimental import pallas as pl
from jax.experimental.pallas import tpu as pltpu

H = 128
MB = 128
JB = 256
MG = 8


def _pair_kernel(a_ref, b_ref, w1_ref, b1_ref, w2_ref, b2_ref, out_ref,
                 tsc_ref):
    a = a_ref[...]
    b = b_ref[...]
    w1 = w1_ref[...]
    ha = jax.lax.dot_general(a, w1[:, :H], (((1,), (1,)), ((), ())),
                             preferred_element_type=jnp.float32)
    ha = ha + b1_ref[...]
    hb = jax.lax.dot_general(b, w1[:, H:], (((1,), (1,)), ((), ())),
                             preferred_element_type=jnp.float32)

    w2 = w2_ref[...]
    b2 = b2_ref[0]
    hab = ha.astype(jnp.bfloat16)
    hbb = hb.astype(jnp.bfloat16)
    w2b = w2.astype(jnp.bfloat16)
    for mg in range(0, MB, MG):
        t = jnp.maximum(hab[mg:mg + MG, None, :] + hbb[None, :, :],
                        jnp.bfloat16(0.0))
        s3 = jnp.sum(t * w2b[None, :, :], axis=-1, keepdims=True,
                     dtype=jnp.bfloat16)
        for r in range(MG):
            tsc_ref[:, mg + r:mg + r + 1] = s3[r]
    out_ref[...] = tsc_ref[...].T.astype(jnp.float32) + b2


def kernel(pred_a, pred_b, W1, b1, W2, b2):
    n_m = pred_a.shape[0]
    n_j = pred_b.shape[0]
    b1r = b1.reshape(1, 2 * H)
    grid = (n_m // MB, n_j // JB)
    out = pl.pallas_call(
        _pair_kernel,
        grid=grid,
        in_specs=[
            pl.BlockSpec((MB, H), lambda i, j: (i, 0)),
            pl.BlockSpec((JB, H), lambda i, j: (j, 0)),
            pl.BlockSpec((2 * H, 2 * H), lambda i, j: (0, 0)),
            pl.BlockSpec((1, 2 * H), lambda i, j: (0, 0)),
            pl.BlockSpec((1, 2 * H), lambda i, j: (0, 0)),
            pl.BlockSpec(memory_space=pltpu.SMEM),
        ],
        out_specs=pl.BlockSpec((MB, JB), lambda i, j: (i, j)),
        out_shape=jax.ShapeDtypeStruct((n_m, n_j), jnp.float32),
        scratch_shapes=[pltpu.VMEM((JB, MB), jnp.bfloat16)],
        compiler_params=pltpu.CompilerParams(
            dimension_semantics=("parallel", "arbitrary"),
            vmem_limit_bytes=100 * 1024 * 1024,
        ),
    )(pred_a, pred_b, W1, b1r, W2, b2)
    return out


# speedup vs baseline: 2.0264x; 1.0449x over previous
import jax
import jax.numpy as jnp
from jax.experimental import pallas as pl
from jax.experimental.pallas import tpu as pltpu

H = 128
MB = 256
JB = 256
MG = 8


def _pair_kernel(a_ref, b_ref, w1_ref, b1_ref, w2_ref, b2_ref, out_ref,
                 tsc_ref):
    a = a_ref[...]
    b = b_ref[...]
    w1 = w1_ref[...]
    ha = jax.lax.dot_general(a, w1[:, :H], (((1,), (1,)), ((), ())),
                             preferred_element_type=jnp.float32)
    ha = ha + b1_ref[...]
    hb = jax.lax.dot_general(b, w1[:, H:], (((1,), (1,)), ((), ())),
                             preferred_element_type=jnp.float32)

    w2 = w2_ref[...]
    b2 = b2_ref[0]
    hab = ha.astype(jnp.bfloat16)
    hbb = hb.astype(jnp.bfloat16)
    w2b = w2.astype(jnp.bfloat16)
    for mg in range(0, MB, MG):
        t = jnp.maximum(hab[mg:mg + MG, None, :] + hbb[None, :, :],
                        jnp.bfloat16(0.0))
        s3 = jnp.sum(t * w2b[None, :, :], axis=-1, keepdims=True,
                     dtype=jnp.bfloat16)
        for r in range(MG):
            tsc_ref[:, mg + r:mg + r + 1] = s3[r]
    out_ref[...] = tsc_ref[...].T.astype(jnp.float32) + b2


def kernel(pred_a, pred_b, W1, b1, W2, b2):
    n_m = pred_a.shape[0]
    n_j = pred_b.shape[0]
    b1r = b1.reshape(1, 2 * H)
    grid = (n_m // MB, n_j // JB)
    out = pl.pallas_call(
        _pair_kernel,
        grid=grid,
        in_specs=[
            pl.BlockSpec((MB, H), lambda i, j: (i, 0)),
            pl.BlockSpec((JB, H), lambda i, j: (j, 0)),
            pl.BlockSpec((2 * H, 2 * H), lambda i, j: (0, 0)),
            pl.BlockSpec((1, 2 * H), lambda i, j: (0, 0)),
            pl.BlockSpec((1, 2 * H), lambda i, j: (0, 0)),
            pl.BlockSpec(memory_space=pltpu.SMEM),
        ],
        out_specs=pl.BlockSpec((MB, JB), lambda i, j: (i, j)),
        out_shape=jax.ShapeDtypeStruct((n_m, n_j), jnp.float32),
        scratch_shapes=[pltpu.VMEM((JB, MB), jnp.bfloat16)],
        compiler_params=pltpu.CompilerParams(
            dimension_semantics=("parallel", "arbitrary"),
            vmem_limit_bytes=100 * 1024 * 1024,
        ),
    )(pred_a, pred_b, W1, b1r, W2, b2)
    return out


# speedup vs baseline: 2.0962x; 1.0344x over previous
import jax
import jax.numpy as jnp
from jax.experimental import pallas as pl
from jax.experimental.pallas import tpu as pltpu

H = 128
MB = 1024
JB = 256
MG = 8


def _pair_kernel(a_ref, b_ref, w1_ref, b1_ref, w2_ref, b2_ref, out_ref,
                 tsc_ref):
    a = a_ref[...]
    b = b_ref[...]
    w1 = w1_ref[...]
    ha = jax.lax.dot_general(a, w1[:, :H], (((1,), (1,)), ((), ())),
                             preferred_element_type=jnp.float32)
    ha = ha + b1_ref[...]
    hb = jax.lax.dot_general(b, w1[:, H:], (((1,), (1,)), ((), ())),
                             preferred_element_type=jnp.float32)

    w2 = w2_ref[...]
    b2 = b2_ref[0]
    hab = ha.astype(jnp.bfloat16)
    hbb = hb.astype(jnp.bfloat16)
    w2b = w2.astype(jnp.bfloat16)
    for mg in range(0, MB, MG):
        t = jnp.maximum(hab[mg:mg + MG, None, :] + hbb[None, :, :],
                        jnp.bfloat16(0.0))
        s3 = jnp.sum(t * w2b[None, :, :], axis=-1, keepdims=True,
                     dtype=jnp.bfloat16)
        for r in range(MG):
            tsc_ref[:, mg + r:mg + r + 1] = s3[r]
    out_ref[...] = tsc_ref[...].T.astype(jnp.float32) + b2


def kernel(pred_a, pred_b, W1, b1, W2, b2):
    n_m = pred_a.shape[0]
    n_j = pred_b.shape[0]
    b1r = b1.reshape(1, 2 * H)
    grid = (n_m // MB, n_j // JB)
    out = pl.pallas_call(
        _pair_kernel,
        grid=grid,
        in_specs=[
            pl.BlockSpec((MB, H), lambda i, j: (i, 0)),
            pl.BlockSpec((JB, H), lambda i, j: (j, 0)),
            pl.BlockSpec((2 * H, 2 * H), lambda i, j: (0, 0)),
            pl.BlockSpec((1, 2 * H), lambda i, j: (0, 0)),
            pl.BlockSpec((1, 2 * H), lambda i, j: (0, 0)),
            pl.BlockSpec(memory_space=pltpu.SMEM),
        ],
        out_specs=pl.BlockSpec((MB, JB), lambda i, j: (i, j)),
        out_shape=jax.ShapeDtypeStruct((n_m, n_j), jnp.float32),
        scratch_shapes=[pltpu.VMEM((JB, MB), jnp.bfloat16)],
        compiler_params=pltpu.CompilerParams(
            dimension_semantics=("parallel", "arbitrary"),
            vmem_limit_bytes=100 * 1024 * 1024,
        ),
    )(pred_a, pred_b, W1, b1r, W2, b2)
    return out
